# baseline (speedup 1.0000x reference)
import functools

import jax
import jax.numpy as jnp
from jax import lax
from jax.experimental import pallas as pl
from jax.experimental.pallas import tpu as pltpu
from jax.experimental.pallas import tpu_sc as plsc

B = 4
S = 8192
D = 1024

_INFO = plsc.get_sparse_core_info()
NC = _INFO.num_cores
NS = _INFO.num_subcores
NW = NC * NS
LANES = 16

ROWS_PER_W = S // NW
C = 32
CL = C * D
N_CHUNKS = ROWS_PER_W // C
BATCH_STRIDE = S * D


def _body(in_hbm, tab_hbm, out_hbm, tbuf, iobuf):
    cid = lax.axis_index("c")
    sid = lax.axis_index("s")
    wid = sid * NC + cid
    w_base = wid * (ROWS_PER_W * D)

    for chunk in range(N_CHUNKS):
        tbase = w_base + chunk * CL
        pltpu.sync_copy(tab_hbm.at[pl.ds(tbase, CL)], tbuf)
        for b in range(B):
            io_base = b * BATCH_STRIDE + tbase
            pltpu.sync_copy(in_hbm.at[pl.ds(io_base, CL)], iobuf)

            @plsc.parallel_loop(0, CL, step=LANES, unroll=8)
            def _add(i):
                plsc.addupdate(iobuf.at[pl.ds(i, LANES)], tbuf[pl.ds(i, LANES)])

            pltpu.sync_copy(iobuf, out_hbm.at[pl.ds(io_base, CL)])


@jax.jit
def kernel(input_embeddings, emb_table):
    inp = input_embeddings.reshape(B * S * D)
    tab = emb_table.reshape(S * D)
    kfn = pl.kernel(
        _body,
        out_type=jax.ShapeDtypeStruct((B * S * D,), jnp.float32),
        mesh=plsc.VectorSubcoreMesh(core_axis_name="c", subcore_axis_name="s"),
        scratch_types=[
            pltpu.VMEM((CL,), jnp.float32),
            pltpu.VMEM((CL,), jnp.float32),
        ],
    )
    out = kfn(inp, tab)
    return out.reshape(B, S, D)


# speedup vs baseline: 2.0673x; 2.0673x over previous
import functools

import jax
import jax.numpy as jnp
from jax import lax
from jax.experimental import pallas as pl
from jax.experimental.pallas import tpu as pltpu
from jax.experimental.pallas import tpu_sc as plsc

B = 4
S = 8192
D = 1024

_INFO = plsc.get_sparse_core_info()
NC = _INFO.num_cores
NS = _INFO.num_subcores
NW = NC * NS
LANES = 16

ROWS_PER_W = S // NW
C = 32
CL = C * D
N_CHUNKS = ROWS_PER_W // C
BATCH_STRIDE = S * D


def _body(in_hbm, tab_hbm, out_hbm, tbuf, iobuf):
    cid = lax.axis_index("c")
    sid = lax.axis_index("s")
    wid = sid * NC + cid
    w_row0 = wid * ROWS_PER_W

    for chunk in range(N_CHUNKS):
        r0 = w_row0 + chunk * C
        pltpu.sync_copy(tab_hbm.at[pl.ds(r0, C)], tbuf)
        for b in range(B):
            pltpu.sync_copy(in_hbm.at[b, pl.ds(r0, C)], iobuf)

            @plsc.parallel_loop(0, C, step=1)
            def _add_row(r):
                @plsc.parallel_loop(0, D, step=LANES, unroll=8)
                def _add(j):
                    plsc.addupdate(
                        iobuf.at[r, pl.ds(j, LANES)], tbuf[r, pl.ds(j, LANES)]
                    )

            pltpu.sync_copy(iobuf, out_hbm.at[b, pl.ds(r0, C)])


@jax.jit
def kernel(input_embeddings, emb_table):
    kfn = pl.kernel(
        _body,
        out_type=jax.ShapeDtypeStruct((B, S, D), jnp.float32),
        mesh=plsc.VectorSubcoreMesh(core_axis_name="c", subcore_axis_name="s"),
        scratch_types=[
            pltpu.VMEM((C, D), jnp.float32),
            pltpu.VMEM((C, D), jnp.float32),
        ],
    )
    return kfn(input_embeddings, emb_table)


# speedup vs baseline: 2.9779x; 1.4405x over previous
import functools

import jax
import jax.numpy as jnp
from jax import lax
from jax.experimental import pallas as pl
from jax.experimental.pallas import tpu as pltpu
from jax.experimental.pallas import tpu_sc as plsc

B = 4
S = 8192
D = 1024

_INFO = plsc.get_sparse_core_info()
NC = _INFO.num_cores
NS = _INFO.num_subcores
NW = NC * NS
LANES = 16

ROWS_PER_W = S // NW
C = 32
CL = C * D
N_CHUNKS = ROWS_PER_W // C
BATCH_STRIDE = S * D


def _add_chunk(iobuf, tbuf):
    @plsc.parallel_loop(0, C, step=1)
    def _add_row(r):
        @plsc.parallel_loop(0, D, step=LANES, unroll=8)
        def _add(j):
            plsc.addupdate(iobuf.at[r, pl.ds(j, LANES)], tbuf[r, pl.ds(j, LANES)])


def _body(in_hbm, tab_hbm, out_hbm, tbuf, io0, io1, sem_in, sem_out, sem_tab):
    cid = lax.axis_index("c")
    sid = lax.axis_index("s")
    wid = sid * NC + cid
    w_row0 = wid * ROWS_PER_W

    ios = (io0, io1)
    steps = [(c, b) for c in range(N_CHUNKS) for b in range(B)]
    n = len(steps)

    def in_copy(s, buf):
        c, b = steps[s]
        r0 = w_row0 + c * C
        return pltpu.async_copy(in_hbm.at[b, pl.ds(r0, C)], buf, sem_in)

    tab_dma = pltpu.async_copy(tab_hbm.at[pl.ds(w_row0, C)], tbuf, sem_tab)
    in_dma = in_copy(0, ios[0])
    out_pending = [None, None]

    for s, (c, b) in enumerate(steps):
        p = s % 2
        if b == 0:
            tab_dma.wait()
        if s + 1 < n:
            if out_pending[1 - p] is not None:
                out_pending[1 - p].wait()
            nxt = in_copy(s + 1, ios[1 - p])
        in_dma.wait()
        in_dma = nxt if s + 1 < n else None
        _add_chunk(ios[p], tbuf)
        if b == B - 1 and c + 1 < N_CHUNKS:
            tab_dma = pltpu.async_copy(
                tab_hbm.at[pl.ds(w_row0 + (c + 1) * C, C)], tbuf, sem_tab
            )
        r0 = w_row0 + c * C
        out_pending[p] = pltpu.async_copy(ios[p], out_hbm.at[b, pl.ds(r0, C)], sem_out)

    for d in out_pending:
        if d is not None:
            d.wait()


@jax.jit
def kernel(input_embeddings, emb_table):
    kfn = pl.kernel(
        _body,
        out_type=jax.ShapeDtypeStruct((B, S, D), jnp.float32),
        mesh=plsc.VectorSubcoreMesh(core_axis_name="c", subcore_axis_name="s"),
        scratch_types=[
            pltpu.VMEM((C, D), jnp.float32),
            pltpu.VMEM((C, D), jnp.float32),
            pltpu.VMEM((C, D), jnp.float32),
            pltpu.SemaphoreType.DMA,
            pltpu.SemaphoreType.DMA,
            pltpu.SemaphoreType.DMA,
        ],
    )
    return kfn(input_embeddings, emb_table)
